# Initial kernel scaffold; baseline (speedup 1.0000x reference)
import jax
import jax.numpy as jnp
from jax.experimental import pallas as pl


def kernel(input, weight_indices, weight_values, bias_indices, bias_values):
    raise NotImplementedError("write your pallas kernel here")


# speedup vs baseline: 1.8861x; 1.8861x over previous
import functools

import jax
import jax.numpy as jnp
from jax import lax
from jax.experimental import pallas as pl
from jax.experimental.pallas import tpu as pltpu
from jax.experimental.pallas import tpu_sc as plsc

B = 256
F = 16384
L = 16
NC = 2
NS = 16
NW = NC * NS
B_PER_W = B // NW
N_B = 2
N_BATCH = B_PER_W // N_B
C = 8192


def _body(n_chunks, inp_hbm, packed_hbm, vals_hbm, bias_idx_hbm,
          bias_val_hbm, out_hbm, inp0_v, inp1_v, acc0_v, acc1_v,
          pk_v, vl_v, bi_v, bv_v):
  inp_refs = (inp0_v, inp1_v)
  acc_refs = (acc0_v, acc1_v)
  cid = lax.axis_index("c")
  sid = lax.axis_index("s")
  wid = sid * NC + cid

  pltpu.sync_copy(bias_idx_hbm, bi_v)
  pltpu.sync_copy(bias_val_hbm, bv_v)
  n_bias_vec = bi_v.shape[0] // L

  def batch_body(batch, _):
    b0 = wid * B_PER_W + batch * N_B

    for j in range(N_B):
      pltpu.sync_copy(inp_hbm.at[b0 + j], inp_refs[j])

      def zero_body(i, _, j=j):
        acc_refs[j][pl.ds(i * L, L)] = jnp.zeros((L,), jnp.float32)
        return 0
      lax.fori_loop(0, F // L, zero_body, 0)

      def bias_body(i, _, j=j):
        idx = bi_v[pl.ds(i * L, L)]
        val = bv_v[pl.ds(i * L, L)]
        plsc.addupdate_scatter(acc_refs[j], [idx], val)
        return 0
      lax.fori_loop(0, n_bias_vec, bias_body, 0)

    def chunk_body(ch, _):
      pltpu.sync_copy(packed_hbm.at[pl.ds(ch * C, C)], pk_v)
      pltpu.sync_copy(vals_hbm.at[pl.ds(ch * C, C)], vl_v)

      def inner(i, _):
        pk = pk_v[pl.ds(i * L, L)]
        v = vl_v[pl.ds(i * L, L)]
        r = pk >> 14
        c = pk & (F - 1)
        for j in range(N_B):
          x = plsc.load_gather(inp_refs[j], [c])
          plsc.addupdate_scatter(acc_refs[j], [r], x * v)
        return 0
      lax.fori_loop(0, C // L, inner, 0)
      return 0
    lax.fori_loop(0, n_chunks, chunk_body, 0)

    for j in range(N_B):
      pltpu.sync_copy(acc_refs[j], out_hbm.at[b0 + j])
    return 0
  lax.fori_loop(0, N_BATCH, batch_body, 0)


def kernel(input, weight_indices, weight_values, bias_indices, bias_values):
  rows = weight_indices[0].astype(jnp.int32)
  cols = weight_indices[1].astype(jnp.int32)
  packed = rows * F + cols
  vals = weight_values.astype(jnp.float32)

  nnz = packed.shape[0]
  n_chunks = -(-nnz // C)
  pad = n_chunks * C - nnz
  packed = jnp.concatenate([packed, jnp.zeros((pad,), jnp.int32)])
  vals = jnp.concatenate([vals, jnp.zeros((pad,), jnp.float32)])

  bias_idx = bias_indices.astype(jnp.int32)
  bn = bias_idx.shape[0]
  bias_pad = -(-bn // L) * L - bn
  bias_idx = jnp.concatenate([bias_idx, jnp.zeros((bias_pad,), jnp.int32)])
  bias_val = jnp.concatenate(
      [bias_values.astype(jnp.float32), jnp.zeros((bias_pad,), jnp.float32)])

  mesh = plsc.VectorSubcoreMesh(core_axis_name="c", subcore_axis_name="s")
  run = pl.kernel(
      functools.partial(_body, n_chunks),
      out_type=jax.ShapeDtypeStruct((B, F), jnp.float32),
      mesh=mesh,
      compiler_params=pltpu.CompilerParams(needs_layout_passes=False),
      scratch_types=[
          pltpu.VMEM((F,), jnp.float32),
          pltpu.VMEM((F,), jnp.float32),
          pltpu.VMEM((F,), jnp.float32),
          pltpu.VMEM((F,), jnp.float32),
          pltpu.VMEM((C,), jnp.int32),
          pltpu.VMEM((C,), jnp.float32),
          pltpu.VMEM((bias_idx.shape[0],), jnp.int32),
          pltpu.VMEM((bias_idx.shape[0],), jnp.float32),
      ],
  )
  return run(input, packed, vals, bias_idx, bias_val)


# speedup vs baseline: 7.1043x; 3.7666x over previous
import functools

import jax
import jax.numpy as jnp
from jax import lax
from jax.experimental import pallas as pl
from jax.experimental.pallas import tpu as pltpu
from jax.experimental.pallas import tpu_sc as plsc

B = 256
F = 16384
L = 16
NC = 2
NS = 16
NW = NC * NS
B_PER_W = B // NW
N_B = 2
N_BATCH = B_PER_W // N_B
C = 4096
UNROLL = 8


def _body(n_chunks, inp_hbm, packed_hbm, vals_hbm, bias_idx_hbm,
          bias_val_hbm, out_hbm, inp0_v, inp1_v, acc0_v, acc1_v,
          pk0_v, vl0_v, pk1_v, vl1_v, bi_v, bv_v, sem0, sem1):
  inp_refs = (inp0_v, inp1_v)
  acc_refs = (acc0_v, acc1_v)
  pk_refs = (pk0_v, pk1_v)
  vl_refs = (vl0_v, vl1_v)
  sems = (sem0, sem1)
  cid = lax.axis_index("c")
  sid = lax.axis_index("s")
  wid = sid * NC + cid

  pltpu.sync_copy(bias_idx_hbm, bi_v)
  pltpu.sync_copy(bias_val_hbm, bv_v)
  n_bias_vec = bi_v.shape[0] // L

  def start_fetch(ch, slot):
    pltpu.async_copy(packed_hbm.at[pl.ds(ch * C, C)], pk_refs[slot],
                     sems[slot])
    pltpu.async_copy(vals_hbm.at[pl.ds(ch * C, C)], vl_refs[slot],
                     sems[slot])

  def wait_fetch(ch, slot):
    pltpu.make_async_copy(packed_hbm.at[pl.ds(ch * C, C)], pk_refs[slot],
                          sems[slot]).wait()
    pltpu.make_async_copy(vals_hbm.at[pl.ds(ch * C, C)], vl_refs[slot],
                          sems[slot]).wait()

  def process(slot):
    pk_ref, vl_ref = pk_refs[slot], vl_refs[slot]

    @plsc.parallel_loop(0, C, step=L, unroll=UNROLL)
    def _(off):
      pk = pk_ref[pl.ds(off, L)]
      v = vl_ref[pl.ds(off, L)]
      r = pk >> 14
      c = pk & (F - 1)
      for j in range(N_B):
        x = plsc.load_gather(inp_refs[j], [c])
        plsc.addupdate_scatter(acc_refs[j], [r], x * v)

  n_pairs = n_chunks // 2
  assert n_pairs * 2 == n_chunks

  def batch_body(batch, _):
    b0 = wid * B_PER_W + batch * N_B

    start_fetch(0, 0)

    for j in range(N_B):
      pltpu.sync_copy(inp_hbm.at[b0 + j], inp_refs[j])

      @plsc.parallel_loop(0, F, step=L, unroll=UNROLL)
      def _(off, j=j):
        acc_refs[j][pl.ds(off, L)] = jnp.zeros((L,), jnp.float32)

      def bias_body(i, _, j=j):
        idx = bi_v[pl.ds(i * L, L)]
        val = bv_v[pl.ds(i * L, L)]
        plsc.addupdate_scatter(acc_refs[j], [idx], val)
        return 0
      lax.fori_loop(0, n_bias_vec, bias_body, 0)

    def pair_body(p, _):
      ch0 = 2 * p
      start_fetch(ch0 + 1, 1)
      wait_fetch(ch0, 0)
      process(0)

      @pl.when(p < n_pairs - 1)
      def _():
        start_fetch(ch0 + 2, 0)
      wait_fetch(ch0 + 1, 1)
      process(1)
      return 0
    lax.fori_loop(0, n_pairs, pair_body, 0)

    for j in range(N_B):
      pltpu.sync_copy(acc_refs[j], out_hbm.at[b0 + j])
    return 0
  lax.fori_loop(0, N_BATCH, batch_body, 0)


def kernel(input, weight_indices, weight_values, bias_indices, bias_values):
  rows = weight_indices[0].astype(jnp.int32)
  cols = weight_indices[1].astype(jnp.int32)
  packed = rows * F + cols
  vals = weight_values.astype(jnp.float32)

  nnz = packed.shape[0]
  n_chunks = -(-nnz // C)
  pad = n_chunks * C - nnz
  packed = jnp.concatenate([packed, jnp.zeros((pad,), jnp.int32)])
  vals = jnp.concatenate([vals, jnp.zeros((pad,), jnp.float32)])

  bias_idx = bias_indices.astype(jnp.int32)
  bn = bias_idx.shape[0]
  bias_pad = -(-bn // L) * L - bn
  bias_idx = jnp.concatenate([bias_idx, jnp.zeros((bias_pad,), jnp.int32)])
  bias_val = jnp.concatenate(
      [bias_values.astype(jnp.float32), jnp.zeros((bias_pad,), jnp.float32)])

  mesh = plsc.VectorSubcoreMesh(core_axis_name="c", subcore_axis_name="s")
  run = pl.kernel(
      functools.partial(_body, n_chunks),
      out_type=jax.ShapeDtypeStruct((B, F), jnp.float32),
      mesh=mesh,
      compiler_params=pltpu.CompilerParams(needs_layout_passes=False),
      scratch_types=[
          pltpu.VMEM((F,), jnp.float32),
          pltpu.VMEM((F,), jnp.float32),
          pltpu.VMEM((F,), jnp.float32),
          pltpu.VMEM((F,), jnp.float32),
          pltpu.VMEM((C,), jnp.int32),
          pltpu.VMEM((C,), jnp.float32),
          pltpu.VMEM((C,), jnp.int32),
          pltpu.VMEM((C,), jnp.float32),
          pltpu.VMEM((bias_idx.shape[0],), jnp.int32),
          pltpu.VMEM((bias_idx.shape[0],), jnp.float32),
          pltpu.SemaphoreType.DMA,
          pltpu.SemaphoreType.DMA,
      ],
  )
  return run(input, packed, vals, bias_idx, bias_val)


# speedup vs baseline: 7.4752x; 1.0522x over previous
import functools

import jax
import jax.numpy as jnp
from jax import lax
from jax.experimental import pallas as pl
from jax.experimental.pallas import tpu as pltpu
from jax.experimental.pallas import tpu_sc as plsc

B = 256
F = 16384
L = 16
NC = 2
NS = 16
NW = NC * NS
B_PER_W = B // NW
BATCHES = ((0, 1, 2), (3, 4, 5), (6, 7))
C = 4096
UNROLL = 8


def _body(n_chunks, inp_hbm, packed_hbm, vals_hbm, bias_idx_hbm,
          bias_val_hbm, out_hbm, inp0_v, inp1_v, inp2_v,
          acc0_v, acc1_v, acc2_v, pk0_v, vl0_v, pk1_v, vl1_v,
          bi_v, bv_v, sem0, sem1):
  inp_refs = (inp0_v, inp1_v, inp2_v)
  acc_refs = (acc0_v, acc1_v, acc2_v)
  pk_refs = (pk0_v, pk1_v)
  vl_refs = (vl0_v, vl1_v)
  sems = (sem0, sem1)
  cid = lax.axis_index("c")
  sid = lax.axis_index("s")
  wid = sid * NC + cid
  b_base = wid * B_PER_W

  pltpu.sync_copy(bias_idx_hbm, bi_v)
  pltpu.sync_copy(bias_val_hbm, bv_v)
  n_bias_vec = bi_v.shape[0] // L

  def start_fetch(ch, slot):
    pltpu.async_copy(packed_hbm.at[pl.ds(ch * C, C)], pk_refs[slot],
                     sems[slot])
    pltpu.async_copy(vals_hbm.at[pl.ds(ch * C, C)], vl_refs[slot],
                     sems[slot])

  def wait_fetch(ch, slot):
    pltpu.make_async_copy(packed_hbm.at[pl.ds(ch * C, C)], pk_refs[slot],
                          sems[slot]).wait()
    pltpu.make_async_copy(vals_hbm.at[pl.ds(ch * C, C)], vl_refs[slot],
                          sems[slot]).wait()

  n_pairs = n_chunks // 2
  assert n_pairs * 2 == n_chunks

  for group in BATCHES:
    nb = len(group)

    def process(slot, nb=nb):
      pk_ref, vl_ref = pk_refs[slot], vl_refs[slot]

      @plsc.parallel_loop(0, C, step=L, unroll=UNROLL)
      def _(off):
        pk = pk_ref[pl.ds(off, L)]
        v = vl_ref[pl.ds(off, L)]
        r = pk >> 14
        c = pk & (F - 1)
        for j in range(nb):
          x = plsc.load_gather(inp_refs[j], [c])
          plsc.addupdate_scatter(acc_refs[j], [r], x * v)

    start_fetch(0, 0)

    for j, db in enumerate(group):
      pltpu.sync_copy(inp_hbm.at[b_base + db], inp_refs[j])

      @plsc.parallel_loop(0, F, step=L, unroll=UNROLL)
      def _(off, j=j):
        acc_refs[j][pl.ds(off, L)] = jnp.zeros((L,), jnp.float32)

      def bias_body(i, _, j=j):
        idx = bi_v[pl.ds(i * L, L)]
        val = bv_v[pl.ds(i * L, L)]
        plsc.addupdate_scatter(acc_refs[j], [idx], val)
        return 0
      lax.fori_loop(0, n_bias_vec, bias_body, 0)

    def pair_body(p, _, process=process):
      ch0 = 2 * p
      start_fetch(ch0 + 1, 1)
      wait_fetch(ch0, 0)
      process(0)

      @pl.when(p < n_pairs - 1)
      def _():
        start_fetch(ch0 + 2, 0)
      wait_fetch(ch0 + 1, 1)
      process(1)
      return 0
    lax.fori_loop(0, n_pairs, pair_body, 0)

    for j, db in enumerate(group):
      pltpu.sync_copy(acc_refs[j], out_hbm.at[b_base + db])


def kernel(input, weight_indices, weight_values, bias_indices, bias_values):
  rows = weight_indices[0].astype(jnp.int32)
  cols = weight_indices[1].astype(jnp.int32)
  packed = rows * F + cols
  vals = weight_values.astype(jnp.float32)

  nnz = packed.shape[0]
  n_chunks = -(-nnz // C)
  if n_chunks % 2:
    n_chunks += 1
  pad = n_chunks * C - nnz
  packed = jnp.concatenate([packed, jnp.zeros((pad,), jnp.int32)])
  vals = jnp.concatenate([vals, jnp.zeros((pad,), jnp.float32)])

  bias_idx = bias_indices.astype(jnp.int32)
  bn = bias_idx.shape[0]
  bias_pad = -(-bn // L) * L - bn
  bias_idx = jnp.concatenate([bias_idx, jnp.zeros((bias_pad,), jnp.int32)])
  bias_val = jnp.concatenate(
      [bias_values.astype(jnp.float32), jnp.zeros((bias_pad,), jnp.float32)])

  mesh = plsc.VectorSubcoreMesh(core_axis_name="c", subcore_axis_name="s")
  run = pl.kernel(
      functools.partial(_body, n_chunks),
      out_type=jax.ShapeDtypeStruct((B, F), jnp.float32),
      mesh=mesh,
      compiler_params=pltpu.CompilerParams(needs_layout_passes=False),
      scratch_types=[
          pltpu.VMEM((F,), jnp.float32),
          pltpu.VMEM((F,), jnp.float32),
          pltpu.VMEM((F,), jnp.float32),
          pltpu.VMEM((F,), jnp.float32),
          pltpu.VMEM((F,), jnp.float32),
          pltpu.VMEM((F,), jnp.float32),
          pltpu.VMEM((C,), jnp.int32),
          pltpu.VMEM((C,), jnp.float32),
          pltpu.VMEM((C,), jnp.int32),
          pltpu.VMEM((C,), jnp.float32),
          pltpu.VMEM((bias_idx.shape[0],), jnp.int32),
          pltpu.VMEM((bias_idx.shape[0],), jnp.float32),
          pltpu.SemaphoreType.DMA,
          pltpu.SemaphoreType.DMA,
      ],
  )
  return run(input, packed, vals, bias_idx, bias_val)
